# baseline (speedup 1.0000x reference)
import functools

import jax
import jax.numpy as jnp
from jax import lax
from jax.experimental import pallas as pl
from jax.experimental.pallas import tpu as pltpu
from jax.experimental.pallas import tpu_sc as plsc

_B = 16384
_D = 128
_EPS = 1e-5
_CHUNK = 128
_NF = _D // 16


def _sc_gather_stats(table, idx2d):
    info = plsc.get_sparse_core_info()
    nc, ns = info.num_cores, info.num_subcores
    nw = nc * ns
    bpw = _B // nw
    chunks = bpw // _CHUNK

    mesh = plsc.VectorSubcoreMesh(core_axis_name="c", subcore_axis_name="s")

    @functools.partial(
        pl.kernel,
        mesh=mesh,
        out_type=(
            jax.ShapeDtypeStruct((_B, _D), jnp.float32),
            jax.ShapeDtypeStruct((2, nw, _D), jnp.float32),
        ),
        scratch_types=[
            pltpu.VMEM((chunks, _CHUNK), jnp.int32),
            pltpu.VMEM((bpw, _D), jnp.float32),
            pltpu.VMEM((2, _D), jnp.float32),
            pltpu.SemaphoreType.DMA,
            pltpu.SemaphoreType.DMA,
        ],
    )
    def gather_kernel(table_hbm, idx_hbm, out_hbm, part_hbm,
                      idx_v, rows_v, part_v, sem_in, sem_out):
        wid = lax.axis_index("s") * nc + lax.axis_index("c")
        base = wid * bpw
        pltpu.sync_copy(idx_hbm.at[pl.ds(wid * chunks, chunks)], idx_v)
        gathers = [
            pltpu.async_copy(
                table_hbm.at[idx_v.at[j]],
                rows_v.at[pl.ds(j * _CHUNK, _CHUNK)],
                sem_in,
            )
            for j in range(chunks)
        ]

        zeros = tuple(jnp.zeros((16,), jnp.float32) for _ in range(_NF))
        sums, sqs = zeros, zeros
        writes = []
        for j in range(chunks):
            gathers[j].wait()

            def row_body(r, carry):
                s, q = carry
                ns_, nq_ = [], []
                for f in range(_NF):
                    x = rows_v[r, pl.ds(f * 16, 16)]
                    ns_.append(s[f] + x)
                    nq_.append(q[f] + x * x)
                return (tuple(ns_), tuple(nq_))

            sums, sqs = lax.fori_loop(
                j * _CHUNK, (j + 1) * _CHUNK, row_body, (sums, sqs))
            writes.append(
                pltpu.async_copy(
                    rows_v.at[pl.ds(j * _CHUNK, _CHUNK)],
                    out_hbm.at[pl.ds(base + j * _CHUNK, _CHUNK)],
                    sem_out,
                )
            )

        for f in range(_NF):
            part_v[0, pl.ds(f * 16, 16)] = sums[f]
            part_v[1, pl.ds(f * 16, 16)] = sqs[f]
        pltpu.sync_copy(part_v.at[0], part_hbm.at[0, wid])
        pltpu.sync_copy(part_v.at[1], part_hbm.at[1, wid])
        for w in writes:
            w.wait()

    return gather_kernel(table, idx2d)


def _tc_affine(x, partials, gamma, beta, nw):
    steps = 1
    rows = _B // steps

    def body(part_ref, g_ref, b_ref, x_ref, o_ref):
        mean = jnp.sum(part_ref[0], axis=0) / _B
        ex2 = jnp.sum(part_ref[1], axis=0) / _B
        var = ex2 - mean * mean
        scale = g_ref[0] * lax.rsqrt(var + _EPS)
        bias = b_ref[0] - mean * scale
        o_ref[...] = x_ref[...] * scale + bias

    return pl.pallas_call(
        body,
        grid=(steps,),
        in_specs=[
            pl.BlockSpec((2, nw, _D), lambda i: (0, 0, 0)),
            pl.BlockSpec((1, _D), lambda i: (0, 0)),
            pl.BlockSpec((1, _D), lambda i: (0, 0)),
            pl.BlockSpec((rows, _D), lambda i: (i, 0)),
        ],
        out_specs=pl.BlockSpec((rows, _D), lambda i: (i, 0)),
        out_shape=jax.ShapeDtypeStruct((_B, _D), jnp.float32),
    )(partials, gamma.reshape(1, _D), beta.reshape(1, _D), x)


def kernel(nodes, table, gamma, beta):
    idx2d = nodes.astype(jnp.int32).reshape(_B // _CHUNK, _CHUNK)
    gathered, partials = _sc_gather_stats(table, idx2d)
    nw = partials.shape[1]
    return _tc_affine(gathered, partials, gamma, beta, nw)


# speedup vs baseline: 1.0464x; 1.0464x over previous
import functools

import jax
import jax.numpy as jnp
from jax import lax
from jax.experimental import pallas as pl
from jax.experimental.pallas import tpu as pltpu
from jax.experimental.pallas import tpu_sc as plsc

_B = 16384
_D = 128
_EPS = 1e-5
_CHUNK = 128
_NF = _D // 16


def _sc_gather_stats(table, idx2d):
    info = plsc.get_sparse_core_info()
    nc, ns = info.num_cores, info.num_subcores
    nw = nc * ns
    bpw = _B // nw
    chunks = bpw // _CHUNK

    mesh = plsc.VectorSubcoreMesh(core_axis_name="c", subcore_axis_name="s")

    @functools.partial(
        pl.kernel,
        mesh=mesh,
        out_type=(
            jax.ShapeDtypeStruct((_B, _D), jnp.float32),
            jax.ShapeDtypeStruct((2, nw, _D), jnp.float32),
        ),
        scratch_types=[
            pltpu.VMEM((chunks, _CHUNK), jnp.int32),
            pltpu.VMEM((bpw, _D), jnp.float32),
            pltpu.VMEM((2, _D), jnp.float32),
            pltpu.SemaphoreType.DMA,
            pltpu.SemaphoreType.DMA,
        ],
    )
    def gather_kernel(table_hbm, idx_hbm, out_hbm, part_hbm,
                      idx_v, rows_v, part_v, sem_in, sem_out):
        wid = lax.axis_index("s") * nc + lax.axis_index("c")
        base = wid * bpw
        pltpu.sync_copy(idx_hbm.at[pl.ds(wid * chunks, chunks)], idx_v)
        gathers = [
            pltpu.async_copy(
                table_hbm.at[idx_v.at[j]],
                rows_v.at[pl.ds(j * _CHUNK, _CHUNK)],
                sem_in,
            )
            for j in range(chunks)
        ]

        zeros = tuple(jnp.zeros((16,), jnp.float32) for _ in range(_NF))
        sums, sqs = zeros, zeros
        writes = []
        for j in range(chunks):
            gathers[j].wait()

            def row_body(r, carry):
                s, q = carry
                ns_, nq_ = [], []
                for f in range(_NF):
                    x = rows_v[r, pl.ds(f * 16, 16)]
                    ns_.append(s[f] + x)
                    nq_.append(q[f] + x * x)
                return (tuple(ns_), tuple(nq_))

            sums, sqs = lax.fori_loop(
                j * _CHUNK, (j + 1) * _CHUNK, row_body, (sums, sqs))
            writes.append(
                pltpu.async_copy(
                    rows_v.at[pl.ds(j * _CHUNK, _CHUNK)],
                    out_hbm.at[pl.ds(base + j * _CHUNK, _CHUNK)],
                    sem_out,
                )
            )

        for f in range(_NF):
            part_v[0, pl.ds(f * 16, 16)] = sums[f]
            part_v[1, pl.ds(f * 16, 16)] = sqs[f]
        pltpu.sync_copy(part_v.at[0], part_hbm.at[0, wid])
        pltpu.sync_copy(part_v.at[1], part_hbm.at[1, wid])
        for w in writes:
            w.wait()

    return gather_kernel(table, idx2d)


def _tc_affine(x, partials, gamma, beta, nw):
    steps = 2
    rows = _B // steps

    def body(part_ref, g_ref, b_ref, x_ref, o_ref):
        mean = jnp.sum(part_ref[0], axis=0) / _B
        ex2 = jnp.sum(part_ref[1], axis=0) / _B
        var = ex2 - mean * mean
        scale = g_ref[0] * lax.rsqrt(var + _EPS)
        bias = b_ref[0] - mean * scale
        o_ref[...] = x_ref[...] * scale + bias

    return pl.pallas_call(
        body,
        grid=(steps,),
        in_specs=[
            pl.BlockSpec((2, nw, _D), lambda i: (0, 0, 0)),
            pl.BlockSpec((1, _D), lambda i: (0, 0)),
            pl.BlockSpec((1, _D), lambda i: (0, 0)),
            pl.BlockSpec((rows, _D), lambda i: (i, 0)),
        ],
        out_specs=pl.BlockSpec((rows, _D), lambda i: (i, 0)),
        out_shape=jax.ShapeDtypeStruct((_B, _D), jnp.float32),
    )(partials, gamma.reshape(1, _D), beta.reshape(1, _D), x)


def kernel(nodes, table, gamma, beta):
    idx2d = nodes.astype(jnp.int32).reshape(_B // _CHUNK, _CHUNK)
    gathered, partials = _sc_gather_stats(table, idx2d)
    nw = partials.shape[1]
    return _tc_affine(gathered, partials, gamma, beta, nw)
